# Initial kernel scaffold; baseline (speedup 1.0000x reference)
import jax
import jax.numpy as jnp
from jax.experimental import pallas as pl


def kernel(A, X, Y, SP, P, PCH, PCL, Op, Val, fu_map, params):
    raise NotImplementedError("write your pallas kernel here")


# speedup vs baseline: 12.7174x; 12.7174x over previous
import functools

import numpy as np

import jax
import jax.numpy as jnp
from jax import lax
from jax.experimental import pallas as pl
from jax.experimental.pallas import tpu as pltpu
from jax.experimental.pallas import tpu_sc as plsc

_H = 128
_NSLOT = 6
_TROWS = _NSLOT * 5 * 256
_NW = 32
_CHUNK = 64
_Z = np.int32(0)


def _build_table(params):
    pa, plg, pm, pf, ps = (params["alu"], params["logic"], params["move"],
                           params["flow"], params["stack"])
    ins = [
        pa["emb_byte"], pa["emb_carry"], pa["emb_op"], pa["W1"],
        pa["b1"].reshape(1, _H),
        plg["emb_op"], plg["W1"], plg["b1"].reshape(1, _H),
        pm["emb_byte"], pm["emb_op"], pm["W1"], pm["b1"].reshape(1, _H),
        pf["emb_op"], pf["W1"], pf["b1"].reshape(1, _H),
        ps["emb_byte"], ps["emb_op"], ps["W1"], ps["b1"].reshape(1, _H),
    ]

    def body(eba, eca, eoa, w1a, b1a, eol, w1l, b1l, ebm, eom, w1m, b1m,
             eof, w1f, b1f, ebs, eos, w1s, b1s, out_ref):
        f32 = jnp.float32
        vi = lax.broadcasted_iota(jnp.int32, (256, 1), 0)
        bits = jnp.concatenate(
            [((vi >> k) & 1).astype(f32) for k in range(8)], axis=1)
        par = (vi & 1).astype(f32)
        EBA, ECA, EOA, W1A = eba[...], eca[...], eoa[...], w1a[...]
        EOL, W1L = eol[...], w1l[...]
        EBM, EOM, W1M = ebm[...], eom[...], w1m[...]
        EOF, W1F = eof[...], w1f[...]
        EBS, EOS, W1S = ebs[...], eos[...], w1s[...]
        B1A, B1L, B1M, B1F, B1S = b1a[...], b1l[...], b1m[...], b1f[...], b1s[...]
        carry_exp = jnp.where((vi & 1) == 1, ECA[1:2, :], ECA[0:1, :])

        def dot(a, b):
            return jnp.dot(a, b, preferred_element_type=f32)

        Z = jnp.zeros((256, _H), f32)
        blocks = [
            dot(EBA, W1A[0:64]), dot(bits, W1L[0:8]),
            dot(EBM, W1M[0:64]), dot(bits, W1F[0:8]), dot(EBS, W1S[0:64]),
            dot(EBA, W1A[64:128]), dot(bits, W1L[8:16]),
            dot(EBM, W1M[64:128]), dot(bits, W1F[8:16]), dot(EBS, W1S[64:128]),
            dot(carry_exp, W1A[128:192]), par * W1L[16:17],
            dot(EBM, W1M[128:192]), dot(bits, W1F[16:24]), dot(EBS, W1S[128:192]),
            dot(EOA, W1A[192:256]) + B1A, dot(EOL, W1L[17:81]) + B1L,
            dot(EBM, W1M[192:256]), dot(bits, W1F[24:32]), dot(bits, W1S[192:200]),
            Z, Z, dot(EOM, W1M[256:320]) + B1M,
            dot(bits, W1F[32:40]), dot(EBS, W1S[200:264]),
            Z, Z, Z, dot(EOF, W1F[40:104]) + B1F, dot(EOS, W1S[264:328]) + B1S,
        ]
        out_ref[...] = jnp.concatenate(blocks, axis=0)

    return pl.pallas_call(
        body,
        out_shape=jax.ShapeDtypeStruct((_TROWS, _H), jnp.float32),
    )(*ins)


def _sc_stage(reg_list, fumap, table):
    Bn = reg_list[0].shape[0]
    BW = Bn // _NW
    nch = BW // _CHUNK
    mesh = plsc.VectorSubcoreMesh(core_axis_name="c", subcore_axis_name="s",
                                  num_cores=2, num_subcores=16)

    @functools.partial(
        pl.kernel,
        out_type=(jax.ShapeDtypeStruct((Bn, _H), jnp.float32),
                  jax.ShapeDtypeStruct((Bn,), jnp.float32)),
        mesh=mesh,
        scratch_types=[
            [pltpu.VMEM((BW,), jnp.int32) for _ in range(9)],
            pltpu.VMEM((256,), jnp.int32),
            pltpu.VMEM((nch * _NSLOT * _CHUNK,), jnp.int32),
            pltpu.VMEM((BW,), jnp.float32),
            [pltpu.VMEM((_CHUNK, _H), jnp.float32) for _ in range(_NSLOT)],
            pltpu.VMEM((_CHUNK, _H), jnp.float32),
            pltpu.SemaphoreType.DMA,
        ],
        compiler_params=pltpu.CompilerParams(needs_layout_passes=False),
    )
    def sc_k(a_h, x_h, y_h, sp_h, p_h, pch_h, pcl_h, op_h, val_h,
             fumap_hbm, table_hbm, h_hbm, fuf_hbm,
             rv, fm, idxv, fub, gb, acc, sem):
        cid = lax.axis_index("c")
        sid = lax.axis_index("s")
        wid = sid * 2 + cid
        base = wid * BW
        regs_hbm = (a_h, x_h, y_h, sp_h, p_h, pch_h, pcl_h, op_h, val_h)
        for j in range(9):
            pltpu.sync_copy(regs_hbm[j].at[pl.ds(base, BW)], rv[j])
        pltpu.sync_copy(fumap_hbm, fm)

        def grp(g, carry):
            off = g * 16
            a = rv[0][pl.ds(off, 16)]
            x = rv[1][pl.ds(off, 16)]
            y = rv[2][pl.ds(off, 16)]
            sp = rv[3][pl.ds(off, 16)]
            p = rv[4][pl.ds(off, 16)]
            pch = rv[5][pl.ds(off, 16)]
            pcl = rv[6][pl.ds(off, 16)]
            op = rv[7][pl.ds(off, 16)]
            val = rv[8][pl.ds(off, 16)]
            fu = plsc.load_gather(fm, [op])
            fub[pl.ds(off, 16)] = fu.astype(jnp.float32)
            fb = fu * 256
            lo = fu <= 1
            i0 = jnp.where(fu == 3, pcl, a)
            i1 = jnp.where(lo, val, jnp.where(fu == 3, pch, x))
            i2 = jnp.where(fu == 2, y, jnp.where(fu == 4, sp, p))
            i3 = jnp.where(lo, op, jnp.where(fu == 4, p, val))
            i4 = jnp.where(fu == 2, op, jnp.where(fu == 3, sp, val))
            i5 = op
            k = g // (_CHUNK // 16)
            col = (g % (_CHUNK // 16)) * 16
            cbase = k * _NSLOT * _CHUNK + col
            idxv[pl.ds(cbase + 0 * _CHUNK, 16)] = i0 + fb
            idxv[pl.ds(cbase + 1 * _CHUNK, 16)] = i1 + fb + 1280
            idxv[pl.ds(cbase + 2 * _CHUNK, 16)] = i2 + fb + 2560
            idxv[pl.ds(cbase + 3 * _CHUNK, 16)] = i3 + fb + 3840
            idxv[pl.ds(cbase + 4 * _CHUNK, 16)] = i4 + fb + 5120
            idxv[pl.ds(cbase + 5 * _CHUNK, 16)] = i5 + fb + 6400
            return carry

        lax.fori_loop(jnp.int32(0), jnp.int32(BW // 16), grp, jnp.int32(0))

        def chunk(k, carry):
            cps = [
                pltpu.async_copy(
                    table_hbm.at[idxv.at[pl.ds(k * (_NSLOT * _CHUNK)
                                               + s * _CHUNK, _CHUNK)]],
                    gb[s], sem)
                for s in range(_NSLOT)
            ]
            for cp in cps:
                cp.wait()

            def vec(i, c2):
                r = i // (_H // 16)
                c = (i % (_H // 16)) * 16
                v = (gb[0][r, pl.ds(c, 16)] + gb[1][r, pl.ds(c, 16)]
                     + gb[2][r, pl.ds(c, 16)] + gb[3][r, pl.ds(c, 16)]
                     + gb[4][r, pl.ds(c, 16)] + gb[5][r, pl.ds(c, 16)])
                acc[r, pl.ds(c, 16)] = jnp.maximum(v, jnp.float32(0.0))
                return c2

            lax.fori_loop(jnp.int32(0), jnp.int32(_CHUNK * _H // 16), vec,
                          jnp.int32(0))
            pltpu.sync_copy(acc, h_hbm.at[pl.ds(base + k * _CHUNK, _CHUNK)])
            return carry

        lax.fori_loop(jnp.int32(0), jnp.int32(nch), chunk, jnp.int32(0))
        pltpu.sync_copy(fub, fuf_hbm.at[pl.ds(base, BW)])

    return sc_k(*reg_list, fumap, table)


def _tc_stage(h, fuf2, w2, b2):
    Bn = h.shape[0]
    blk = 2048
    grid = Bn // blk

    def body(h_ref, fu_ref, w2_ref, b2_ref, out_ref):
        f32 = jnp.float32
        hb = h_ref[...]
        fv = fu_ref[...]
        acc = jnp.zeros((blk, 264), f32)
        for f in range(5):
            m = (fv == jnp.float32(f)).astype(f32)
            acc = acc + jnp.dot(hb * m, w2_ref[f], preferred_element_type=f32)
            acc = acc + m * b2_ref[f][None]
        out_ref[...] = acc

    return pl.pallas_call(
        body,
        grid=(grid,),
        in_specs=[
            pl.BlockSpec((blk, _H), lambda i: (i, _Z)),
            pl.BlockSpec((blk, 1), lambda i: (i, _Z)),
            pl.BlockSpec((5, _H, 264), lambda i: (_Z, _Z, _Z)),
            pl.BlockSpec((5, 264), lambda i: (_Z, _Z)),
        ],
        out_specs=pl.BlockSpec((blk, 264), lambda i: (i, _Z)),
        out_shape=jax.ShapeDtypeStruct((Bn, 264), jnp.float32),
    )(h, fuf2, w2, b2)


def kernel(A, X, Y, SP, P, PCH, PCL, Op, Val, fu_map, params):
    reg_list = [r.astype(jnp.int32) for r in
                (A, X, Y, SP, P, PCH, PCL, Op, Val)]
    fumap32 = fu_map.astype(jnp.int32)
    table = _build_table(params)
    h, fuf = _sc_stage(reg_list, fumap32, table)
    names = ["alu", "logic", "move", "flow", "stack"]
    w2 = jnp.stack([jnp.concatenate([params[n]["W2r"], params[n]["W2f"]],
                                    axis=1) for n in names])
    b2 = jnp.stack([jnp.concatenate([params[n]["b2r"], params[n]["b2f"]])
                    for n in names])
    Bn = reg_list[0].shape[0]
    return _tc_stage(h, fuf.reshape(Bn, 1), w2, b2)
